# baseline (speedup 1.0000x reference)
import functools

import jax
import jax.numpy as jnp
from jax.experimental import pallas as pl
from jax.experimental.pallas import tpu as pltpu


def _dot(a, b):
    return jax.lax.dot_general(
        a, b, (((a.ndim - 1,), (0,)), ((), ())),
        preferred_element_type=jnp.float32)


def _fused_kernel(x_ref, wr_ref, br_ref, wc_ref, bc_ref, w1_ref, b1_ref,
                  w5_ref, b5_ref, w6_ref, b6_ref, w7_ref, b7_ref,
                  o_ref, acc_ref, ph_ref, *, n_steps, bB, nN):
    n = pl.program_id(1)

    x = x_ref[...]
    feat = x.shape[-1]
    h = jnp.maximum(
        _dot(x.reshape(bB * nN, feat), wr_ref[...]) + br_ref[...], 0.0)
    partial = jnp.sum(h.reshape(bB, nN, -1), axis=1)

    @pl.when(n == 0)
    def _init():
        acc_ref[...] = partial
        ph_ref[...] = x[:, 0, -3:]

    @pl.when(n != 0)
    def _accum():
        acc_ref[...] += partial

    @pl.when(n == n_steps - 1)
    def _head():
        g = acc_ref[...]
        c = jnp.maximum(_dot(g, wc_ref[...]) + bc_ref[...], 0.0)
        d = jnp.maximum(_dot(c, w1_ref[...]) + b1_ref[...], 0.0)
        d = _dot(d, w5_ref[...]) + b5_ref[...]
        mv = _dot(d, w6_ref[...]) + b6_ref[...]
        merged = jnp.concatenate([mv, ph_ref[...]], axis=1)
        o_ref[...] = _dot(merged, w7_ref[...]) + b7_ref[...]


@functools.partial(jax.jit, static_argnames=())
def kernel(inputs, W_rule, b_rule, W_conv, b_conv, W1, b1, W5, b5, W6, b6,
           W7, b7):
    B, N, F = inputs.shape
    naf, rule_out = W_rule.shape

    Wp = jnp.concatenate(
        [W_rule, jnp.zeros((F - naf, rule_out), W_rule.dtype)], axis=0)

    bB, nN = 32, 1024
    n_steps = N // nN
    grid = (B // bB, n_steps)

    row = lambda v: v.reshape(1, -1)
    full = lambda a: pl.BlockSpec(a.shape, lambda b, n: (0,) * a.ndim)

    out = pl.pallas_call(
        functools.partial(_fused_kernel, n_steps=n_steps, bB=bB, nN=nN),
        grid=grid,
        in_specs=[
            pl.BlockSpec((bB, nN, F), lambda b, n: (b, n, 0)),
            full(Wp), full(row(b_rule)),
            full(W_conv), full(row(b_conv)),
            full(W1), full(row(b1)),
            full(W5), full(row(b5)),
            full(W6), full(row(b6)),
            full(W7), full(row(b7)),
        ],
        out_specs=pl.BlockSpec((bB, 1), lambda b, n: (b, 0)),
        out_shape=jax.ShapeDtypeStruct((B, 1), jnp.float32),
        scratch_shapes=[
            pltpu.VMEM((bB, rule_out), jnp.float32),
            pltpu.VMEM((bB, 3), jnp.float32),
        ],
        compiler_params=pltpu.CompilerParams(
            dimension_semantics=("parallel", "arbitrary")),
    )(inputs, Wp, row(b_rule), W_conv, row(b_conv), W1, row(b1),
      W5, row(b5), W6, row(b6), W7, row(b7))
    return out


# speedup vs baseline: 1.0750x; 1.0750x over previous
import functools

import jax
import jax.numpy as jnp
from jax.experimental import pallas as pl
from jax.experimental.pallas import tpu as pltpu


def _dot(a, b):
    return jax.lax.dot_general(
        a, b, (((a.ndim - 1,), (0,)), ((), ())),
        preferred_element_type=jnp.float32)


def _fused_kernel(x_ref, wr_ref, br_ref, wc_ref, bc_ref, w1_ref, b1_ref,
                  w5_ref, b5_ref, w6_ref, b6_ref, w7_ref, b7_ref,
                  o_ref, *, bB, rule_out):
    x = x_ref[...]
    b_, m, f2 = x.shape
    h = jnp.maximum(
        _dot(x.reshape(b_ * m, f2), wr_ref[...]) + br_ref[...], 0.0)
    part = jnp.sum(h.reshape(b_, m, 2 * rule_out), axis=1)
    g = part[:, :rule_out] + part[:, rule_out:]

    c = jnp.maximum(_dot(g, wc_ref[...]) + bc_ref[...], 0.0)
    d = jnp.maximum(_dot(c, w1_ref[...]) + b1_ref[...], 0.0)
    d = _dot(d, w5_ref[...]) + b5_ref[...]
    mv = _dot(d, w6_ref[...]) + b6_ref[...]
    ph = x[:, 0, f2 // 2 - 3:f2 // 2]
    merged = jnp.concatenate([mv, ph], axis=1)
    o_ref[...] = _dot(merged, w7_ref[...]) + b7_ref[...]


def kernel(inputs, W_rule, b_rule, W_conv, b_conv, W1, b1, W5, b5, W6, b6,
           W7, b7):
    B, N, F = inputs.shape
    naf, rule_out = W_rule.shape

    Wp = jnp.concatenate(
        [W_rule, jnp.zeros((F - naf, rule_out), W_rule.dtype)], axis=0)
    z = jnp.zeros_like(Wp)
    Wbd = jnp.concatenate(
        [jnp.concatenate([Wp, z], axis=1),
         jnp.concatenate([z, Wp], axis=1)], axis=0)
    bbd = jnp.concatenate([b_rule, b_rule]).reshape(1, -1)

    x2 = inputs.reshape(B, N // 2, 2 * F)

    bB = 32
    grid = (B // bB,)

    row = lambda v: v.reshape(1, -1)
    full = lambda a: pl.BlockSpec(a.shape, lambda b: (0,) * a.ndim)

    out = pl.pallas_call(
        functools.partial(_fused_kernel, bB=bB, rule_out=rule_out),
        grid=grid,
        in_specs=[
            pl.BlockSpec((bB, N // 2, 2 * F), lambda b: (b, 0, 0)),
            full(Wbd), full(bbd),
            full(W_conv), full(row(b_conv)),
            full(W1), full(row(b1)),
            full(W5), full(row(b5)),
            full(W6), full(row(b6)),
            full(W7), full(row(b7)),
        ],
        out_specs=pl.BlockSpec((bB, 1), lambda b: (b, 0)),
        out_shape=jax.ShapeDtypeStruct((B, 1), jnp.float32),
        compiler_params=pltpu.CompilerParams(
            dimension_semantics=("arbitrary",)),
    )(x2, Wbd, bbd, W_conv, row(b_conv), W1, row(b1),
      W5, row(b5), W6, row(b6), W7, row(b7))
    return out
